# baseline (speedup 1.0000x reference)
import functools

import jax
import jax.numpy as jnp
from jax import lax
from jax.experimental import pallas as pl
from jax.experimental.pallas import tpu as pltpu
from jax.experimental.pallas import tpu_sc as plsc

CAP = 100000
NREC = 5000
K = 80
DOBS = 50
BATCH = 4096

ROWS_PER_BLK = 256
NPAD = 5120
WSTART = CAP - NPAD
PADF = NPAD - NREC
D64 = 64


def _mm_thresh_body(c_ref, pc_ref, u_ref, t_ref):
    deltas = jax.lax.dot_general(
        c_ref[...], pc_ref[...], (((1,), (1,)), ((), ())),
        preferred_element_type=jnp.float32)
    s = jax.lax.bitcast_convert_type(deltas, jnp.int32)
    u = jnp.where(s < 0, ~s, s | jnp.int32(-2147483648)).astype(jnp.uint32)
    col = jax.lax.broadcasted_iota(jnp.int32, u.shape, 1)
    u = jnp.where(col >= PADF, u, jnp.uint32(0))
    u_ref[...] = jax.lax.bitcast_convert_type(u, jnp.int32) ^ jnp.int32(-2147483648)
    ones = jnp.ones((NPAD, 8), jnp.float32)
    t = jnp.zeros((ROWS_PER_BLK, 1), jnp.uint32)
    for bit in range(31, 15, -1):
        cand = t | jnp.uint32(1 << bit)
        cmpf = (u >= cand).astype(jnp.float32)
        cnt = jax.lax.dot_general(
            cmpf, ones, (((1,), (0,)), ((), ())),
            preferred_element_type=jnp.float32)[:, :1]
        t = jnp.where(cnt >= K, cand, t)
    ti = jax.lax.bitcast_convert_type(t, jnp.int32) ^ jnp.int32(-2147483648)
    t_ref[...] = ti.reshape(1, 1, ROWS_PER_BLK)


def _deltas_thresh(c, past_c_pad):
    grid = BATCH // ROWS_PER_BLK
    return pl.pallas_call(
        _mm_thresh_body,
        grid=(grid,),
        in_specs=[
            pl.BlockSpec((ROWS_PER_BLK, D64), lambda i: (i, 0)),
            pl.BlockSpec((NPAD, D64), lambda i: (0, 0)),
        ],
        out_specs=[
            pl.BlockSpec((ROWS_PER_BLK, NPAD), lambda i: (i, 0)),
            pl.BlockSpec((1, 1, ROWS_PER_BLK), lambda i: (i, 0, 0)),
        ],
        out_shape=[
            jax.ShapeDtypeStruct((BATCH, NPAD), jnp.int32),
            jax.ShapeDtypeStruct((grid, 1, ROWS_PER_BLK), jnp.int32),
        ],
    )(c, past_c_pad)


MBLK = 1024
NIDS = BATCH + NPAD


def _match_body(ids_ref, wi_ref, out_ref):
    ids = ids_ref[...]
    wi = wi_ref[...]
    m = jnp.full((MBLK, 1), -1, jnp.int32)
    cw = 1024
    for cc in range(BATCH // cw):
        eq = ids == wi[:, cc * cw:(cc + 1) * cw]
        j = jax.lax.broadcasted_iota(jnp.int32, (MBLK, cw), 1) + cc * cw
        m = jnp.maximum(m, jnp.max(jnp.where(eq, j, -1), axis=1,
                                   keepdims=True))
    out_ref[...] = m


def _match(ids_col, wi_row):
    grid = NIDS // MBLK
    return pl.pallas_call(
        _match_body,
        grid=(grid,),
        in_specs=[
            pl.BlockSpec((MBLK, 1), lambda i: (i, 0)),
            pl.BlockSpec((1, BATCH), lambda i: (0, 0)),
        ],
        out_specs=pl.BlockSpec((MBLK, 1), lambda i: (i, 0)),
        out_shape=jax.ShapeDtypeStruct((NIDS, 1), jnp.int32),
    )(ids_col, wi_row)


NW = 32
ROWS_PER_W = BATCH // NW
NVREG = NPAD // 16
LCAP = NPAD + 16


def _vsort16(p):
    v, x = p
    return plsc.sort_key_val(v, x, descending=True)


def _rev(p):
    v, x = p
    return lax.rev(v, (0,)), lax.rev(x, (0,))


def _bitonic_desc(vs):
    if len(vs) == 1:
        return [_vsort16(vs[0])]
    half = len(vs) // 2
    hi, lo = [], []
    for i in range(half):
        av, ax = vs[i]
        bv, bx = vs[i + half]
        sel = av >= bv
        hi.append((jnp.where(sel, av, bv), jnp.where(sel, ax, bx)))
        lo.append((jnp.where(sel, bv, av), jnp.where(sel, bx, ax)))
    return _bitonic_desc(hi) + _bitonic_desc(lo)


def _merge_desc(a, b):
    return _bitonic_desc(a + [_rev(p) for p in reversed(b)])


def _sort128_desc(pairs):
    runs = [[_vsort16(p)] for p in pairs]
    while len(runs) > 1:
        runs = [_merge_desc(runs[i], runs[i + 1])
                for i in range(0, len(runs), 2)]
    return runs[0]


CR_PW = BATCH // NW
PR_PW = NPAD // NW


def _sc_build(mem, write_val, read_idx, m_read, w_tail):
    mesh = plsc.VectorSubcoreMesh(core_axis_name="c", subcore_axis_name="s")

    @functools.partial(
        pl.kernel,
        out_type=[
            jax.ShapeDtypeStruct((BATCH, D64), jnp.float32),
            jax.ShapeDtypeStruct((NPAD, D64), jnp.float32),
        ],
        mesh=mesh,
        compiler_params=pltpu.CompilerParams(
            needs_layout_passes=False, use_tc_tiling_on_sc=False),
        scratch_types=[
            pltpu.VMEM((CR_PW,), jnp.int32),
            pltpu.VMEM((CR_PW,), jnp.int32),
            pltpu.VMEM((CR_PW,), jnp.int32),
            pltpu.VMEM((CR_PW, D64), jnp.float32),
            pltpu.VMEM((CR_PW, D64), jnp.float32),
            pltpu.VMEM((PR_PW,), jnp.int32),
            pltpu.VMEM((PR_PW,), jnp.int32),
            pltpu.VMEM((PR_PW, D64), jnp.float32),
            pltpu.VMEM((PR_PW, D64), jnp.float32),
            pltpu.SemaphoreType.DMA,
            pltpu.SemaphoreType.DMA,
        ],
    )
    def k(mem_hbm, wv_hbm, ridx_hbm, mr_hbm, wt_hbm, c_out, pc_out,
          ridx_v, mr_v, wvi_v, rows_a, rows_b, wt_v, wti_v, rows_c, rows_d,
          sem_a, sem_b):
        wid = lax.axis_index("s") * 2 + lax.axis_index("c")
        iota = lax.iota(jnp.int32, 16)
        base = wid * CR_PW
        base2 = wid * PR_PW
        pltpu.sync_copy(ridx_hbm.at[pl.ds(base, CR_PW)], ridx_v)
        pltpu.sync_copy(mr_hbm.at[pl.ds(base, CR_PW)], mr_v)
        pltpu.sync_copy(wt_hbm.at[pl.ds(base2, PR_PW)], wt_v)
        for z in range(CR_PW // 16):
            mr16 = mr_v[pl.ds(z * 16, 16)]
            dummy = (base + z * 16 + iota) & (BATCH - 1)
            wvi_v[pl.ds(z * 16, 16)] = jnp.where(mr16 >= 0, mr16, dummy)
        for z in range(PR_PW // 16):
            wt16 = wt_v[pl.ds(z * 16, 16)]
            dummy = (base2 + z * 16 + iota) & (BATCH - 1)
            wti_v[pl.ds(z * 16, 16)] = jnp.where(wt16 >= 0, wt16, dummy)
        ha = pltpu.async_copy(mem_hbm.at[ridx_v], rows_a, sem_a)
        hb = pltpu.async_copy(wv_hbm.at[wvi_v], rows_b, sem_b)
        hc = pltpu.async_copy(
            mem_hbm.at[pl.ds(WSTART + base2, PR_PW)], rows_c, sem_a)
        hd = pltpu.async_copy(wv_hbm.at[wti_v], rows_d, sem_b)
        ha.wait()
        hb.wait()

        def cfix(i, carry):
            e = iota + i * 16
            q = lax.shift_right_logical(e, 6)
            rm = e & 63
            b = plsc.load_gather(rows_b, [q, rm])
            mr = plsc.load_gather(mr_v, [q])
            plsc.store_scatter(rows_a, [q, rm], b, mask=mr >= 0)
            return carry

        lax.fori_loop(0, CR_PW * D64 // 16, cfix, jnp.int32(0))
        pltpu.sync_copy(rows_a, c_out.at[pl.ds(base, CR_PW)])

        hc.wait()
        hd.wait()

        def pfix(i, carry):
            e = iota + i * 16
            q = lax.shift_right_logical(e, 6)
            rm = e & 63
            b = plsc.load_gather(rows_d, [q, rm])
            wt = plsc.load_gather(wt_v, [q])
            plsc.store_scatter(rows_c, [q, rm], b, mask=wt >= 0)
            return carry

        lax.fori_loop(0, PR_PW * D64 // 16, pfix, jnp.int32(0))
        pltpu.sync_copy(rows_c, pc_out.at[pl.ds(base2, PR_PW)])

    return k(mem, write_val, read_idx, m_read, w_tail)


def _sc_topk_emit(u, t, tmeta):
    mesh = plsc.VectorSubcoreMesh(core_axis_name="c", subcore_axis_name="s")

    @functools.partial(
        pl.kernel,
        out_type=jax.ShapeDtypeStruct((BATCH, K * 8), jnp.float32),
        mesh=mesh,
        compiler_params=pltpu.CompilerParams(needs_layout_passes=False),
        scratch_types=[
            pltpu.VMEM((NPAD * 8,), jnp.float32),
            pltpu.VMEM((2 * NPAD,), jnp.int32),
            pltpu.VMEM((ROWS_PER_W,), jnp.int32),
            pltpu.VMEM((LCAP,), jnp.int32),
            pltpu.VMEM((LCAP,), jnp.int32),
            pltpu.VMEM((2 * K * 8,), jnp.float32),
            pltpu.SemaphoreType.DMA,
            pltpu.SemaphoreType.DMA,
        ],
    )
    def k(u_hbm, t_hbm, tm_hbm, out_hbm,
          tm_v, u_v, t_v, lv, li, out_v, sem_in, sem_out):
        wid = lax.axis_index("s") * 2 + lax.axis_index("c")
        base = wid * ROWS_PER_W
        pltpu.sync_copy(tm_hbm, tm_v)
        pltpu.sync_copy(t_hbm.at[pl.ds(base, ROWS_PER_W)], t_v)
        pltpu.async_copy(u_hbm.at[base], u_v.at[pl.ds(0, NPAD)], sem_in)
        iota = lax.iota(jnp.int32, 16)
        lane15 = jnp.full((16,), 15, jnp.int32)

        def row_body(r, carry):
            cur = lax.rem(r, 2)
            nxt = 1 - cur

            @pl.when(r + 1 < ROWS_PER_W)
            def _():
                pltpu.async_copy(u_hbm.at[base + r + 1],
                                 u_v.at[pl.ds(nxt * NPAD, NPAD)], sem_in)

            pltpu.make_async_copy(
                u_hbm.at[base], u_v.at[pl.ds(cur * NPAD, NPAD)], sem_in).wait()
            chunk = r // 16
            lane = r - chunk * 16
            tv16 = t_v[pl.ds(chunk * 16, 16)]
            tvec = tv16[jnp.full((16,), lane, jnp.int32)]
            for z in range(8):
                lv[pl.ds(z * 16, 16)] = jnp.full((16,), -2147483648, jnp.int32)
                li[pl.ds(z * 16, 16)] = jnp.zeros((16,), jnp.int32)

            @plsc.parallel_loop(0, NVREG, 1, unroll=4,
                                carry=jnp.zeros((16,), jnp.int32))
            def _vloop(i, nv):
                x = u_v[pl.ds(cur * NPAD + i * 16, 16)]
                m = x >= tvec
                cs = plsc.cumsum(jnp.where(m, jnp.int32(1), jnp.int32(0)))
                pos = nv + cs - 1
                plsc.store_scatter(lv, [pos], x, mask=m)
                plsc.store_scatter(li, [pos], iota + i * 16, mask=m)
                return nv + cs[lane15]

            pairs = [(lv[pl.ds(z * 16, 16)], li[pl.ds(z * 16, 16)])
                     for z in range(8)]
            srt = _sort128_desc(pairs)

            @pl.when(r >= 2)
            def _():
                pltpu.make_async_copy(
                    out_v.at[pl.ds(cur * K * 8, K * 8)],
                    out_hbm.at[base], sem_out).wait()

            for kk in range(K // 16):
                mv, xv = srt[kk]
                uv = lax.bitcast_convert_type(
                    mv ^ jnp.int32(-2147483648), jnp.uint32)
                neg = ~lax.bitcast_convert_type(uv, jnp.int32)
                pos = lax.bitcast_convert_type(
                    uv & jnp.uint32(0x7FFFFFFF), jnp.int32)
                sv = jnp.where(uv >= jnp.uint32(0x80000000), pos, neg)
                val = lax.bitcast_convert_type(sv, jnp.float32)
                j8 = (iota + kk * 16) * 8 + cur * (K * 8)
                plsc.store_scatter(out_v, [j8], val)
                x8 = xv * 8
                for c in range(1, 8):
                    g = plsc.load_gather(tm_v, [x8 + c])
                    plsc.store_scatter(out_v, [j8 + c], g)
            pltpu.async_copy(out_v.at[pl.ds(cur * K * 8, K * 8)],
                             out_hbm.at[base + r], sem_out)
            return carry

        lax.fori_loop(0, ROWS_PER_W, row_body, jnp.int32(0))
        pltpu.make_async_copy(
            out_v.at[pl.ds(0, K * 8)], out_hbm.at[base], sem_out).wait()
        pltpu.make_async_copy(
            out_v.at[pl.ds(K * 8, K * 8)], out_hbm.at[base], sem_out).wait()

    return k(u, t, tmeta)


def kernel(mem, write_val, actions_table, rewards_table, write_idx, read_idx):
    start = CAP - NREC
    ids = jnp.concatenate(
        [read_idx, WSTART + jnp.arange(NPAD, dtype=jnp.int32)])
    m_all = _match(ids.reshape(NIDS, 1), write_idx.reshape(1, BATCH))
    m_all = m_all.reshape(NIDS)
    m_read, w_tail = m_all[:BATCH], m_all[BATCH:]

    mem64 = jnp.pad(mem, ((0, 0), (0, D64 - DOBS)))
    wv64 = jnp.pad(write_val, ((0, 0), (0, D64 - DOBS)))
    c, past_c_pad = _sc_build(mem64, wv64, read_idx, m_read, w_tail)
    u, t = _deltas_thresh(c, past_c_pad)
    t = t.reshape(BATCH)
    tmeta = jnp.concatenate(
        [jnp.zeros((NREC, 1), jnp.float32), actions_table[start:],
         rewards_table[start:]], axis=1)
    tmeta = jnp.concatenate(
        [jnp.zeros((PADF, 8), jnp.float32), tmeta], axis=0).reshape(-1)
    out = _sc_topk_emit(u, t, tmeta)
    return out.reshape(BATCH, K, 8)


# speedup vs baseline: 1.4685x; 1.4685x over previous
import functools

import jax
import jax.numpy as jnp
from jax import lax
from jax.experimental import pallas as pl
from jax.experimental.pallas import tpu as pltpu
from jax.experimental.pallas import tpu_sc as plsc

CAP = 100000
NREC = 5000
K = 80
DOBS = 50
BATCH = 4096

ROWS_PER_BLK = 256
NPAD = 5120
WSTART = CAP - NPAD
PADF = NPAD - NREC
D64 = 64


def _mm_thresh_body(c_ref, pc_ref, u_ref, t_ref):
    deltas = jax.lax.dot_general(
        c_ref[...], pc_ref[...], (((1,), (1,)), ((), ())),
        preferred_element_type=jnp.float32)
    s = jax.lax.bitcast_convert_type(deltas, jnp.int32)
    u = jnp.where(s < 0, ~s, s | jnp.int32(-2147483648)).astype(jnp.uint32)
    col = jax.lax.broadcasted_iota(jnp.int32, u.shape, 1)
    u = jnp.where(col >= PADF, u, jnp.uint32(0))
    u_ref[...] = jax.lax.bitcast_convert_type(u, jnp.int32) ^ jnp.int32(-2147483648)
    t = jnp.zeros((ROWS_PER_BLK, 1), jnp.uint32)
    for bit in range(31, 15, -1):
        cand = t | jnp.uint32(1 << bit)
        cnt = jnp.sum((u >= cand).astype(jnp.int32), axis=1, keepdims=True)
        t = jnp.where(cnt >= K, cand, t)
    ti = jax.lax.bitcast_convert_type(t, jnp.int32) ^ jnp.int32(-2147483648)
    t_ref[...] = ti.reshape(1, 1, ROWS_PER_BLK)


def _deltas_thresh(c, past_c_pad):
    grid = BATCH // ROWS_PER_BLK
    return pl.pallas_call(
        _mm_thresh_body,
        grid=(grid,),
        in_specs=[
            pl.BlockSpec((ROWS_PER_BLK, D64), lambda i: (i, 0)),
            pl.BlockSpec((NPAD, D64), lambda i: (0, 0)),
        ],
        out_specs=[
            pl.BlockSpec((ROWS_PER_BLK, NPAD), lambda i: (i, 0)),
            pl.BlockSpec((1, 1, ROWS_PER_BLK), lambda i: (i, 0, 0)),
        ],
        out_shape=[
            jax.ShapeDtypeStruct((BATCH, NPAD), jnp.int32),
            jax.ShapeDtypeStruct((grid, 1, ROWS_PER_BLK), jnp.int32),
        ],
    )(c, past_c_pad)


MBLK = 1024
NIDS = BATCH + NPAD


def _match_body(ids_ref, wi_ref, out_ref):
    ids = ids_ref[...]
    wi = wi_ref[...]
    m = jnp.full((MBLK, 1), -1, jnp.int32)
    cw = 1024
    for cc in range(BATCH // cw):
        eq = ids == wi[:, cc * cw:(cc + 1) * cw]
        j = jax.lax.broadcasted_iota(jnp.int32, (MBLK, cw), 1) + cc * cw
        m = jnp.maximum(m, jnp.max(jnp.where(eq, j, -1), axis=1,
                                   keepdims=True))
    out_ref[...] = m


def _match(ids_col, wi_row):
    grid = NIDS // MBLK
    return pl.pallas_call(
        _match_body,
        grid=(grid,),
        in_specs=[
            pl.BlockSpec((MBLK, 1), lambda i: (i, 0)),
            pl.BlockSpec((1, BATCH), lambda i: (0, 0)),
        ],
        out_specs=pl.BlockSpec((MBLK, 1), lambda i: (i, 0)),
        out_shape=jax.ShapeDtypeStruct((NIDS, 1), jnp.int32),
    )(ids_col, wi_row)


NW = 32
ROWS_PER_W = BATCH // NW
NVREG = NPAD // 16
LCAP = NPAD + 16


def _vsort16(p):
    v, x = p
    return plsc.sort_key_val(v, x, descending=True)


def _rev(p):
    v, x = p
    return lax.rev(v, (0,)), lax.rev(x, (0,))


def _bitonic_desc(vs):
    if len(vs) == 1:
        return [_vsort16(vs[0])]
    half = len(vs) // 2
    hi, lo = [], []
    for i in range(half):
        av, ax = vs[i]
        bv, bx = vs[i + half]
        sel = av >= bv
        hi.append((jnp.where(sel, av, bv), jnp.where(sel, ax, bx)))
        lo.append((jnp.where(sel, bv, av), jnp.where(sel, bx, ax)))
    return _bitonic_desc(hi) + _bitonic_desc(lo)


def _merge_desc(a, b):
    return _bitonic_desc(a + [_rev(p) for p in reversed(b)])


def _sort128_desc(pairs):
    runs = [[_vsort16(p)] for p in pairs]
    while len(runs) > 1:
        runs = [_merge_desc(runs[i], runs[i + 1])
                for i in range(0, len(runs), 2)]
    return runs[0]


CR_PW = BATCH // NW
PR_PW = NPAD // NW


def _sc_build(mem, write_val, read_idx, m_read, w_tail):
    mesh = plsc.VectorSubcoreMesh(core_axis_name="c", subcore_axis_name="s")

    @functools.partial(
        pl.kernel,
        out_type=[
            jax.ShapeDtypeStruct((BATCH, D64), jnp.float32),
            jax.ShapeDtypeStruct((NPAD, D64), jnp.float32),
        ],
        mesh=mesh,
        compiler_params=pltpu.CompilerParams(
            needs_layout_passes=False, use_tc_tiling_on_sc=False),
        scratch_types=[
            pltpu.VMEM((CR_PW,), jnp.int32),
            pltpu.VMEM((CR_PW,), jnp.int32),
            pltpu.VMEM((CR_PW,), jnp.int32),
            pltpu.VMEM((CR_PW, D64), jnp.float32),
            pltpu.VMEM((CR_PW, D64), jnp.float32),
            pltpu.VMEM((PR_PW,), jnp.int32),
            pltpu.VMEM((PR_PW,), jnp.int32),
            pltpu.VMEM((PR_PW, D64), jnp.float32),
            pltpu.VMEM((PR_PW, D64), jnp.float32),
            pltpu.SemaphoreType.DMA,
            pltpu.SemaphoreType.DMA,
        ],
    )
    def k(mem_hbm, wv_hbm, ridx_hbm, mr_hbm, wt_hbm, c_out, pc_out,
          ridx_v, mr_v, wvi_v, rows_a, rows_b, wt_v, wti_v, rows_c, rows_d,
          sem_a, sem_b):
        wid = lax.axis_index("s") * 2 + lax.axis_index("c")
        iota = lax.iota(jnp.int32, 16)
        base = wid * CR_PW
        base2 = wid * PR_PW
        pltpu.sync_copy(ridx_hbm.at[pl.ds(base, CR_PW)], ridx_v)
        pltpu.sync_copy(mr_hbm.at[pl.ds(base, CR_PW)], mr_v)
        pltpu.sync_copy(wt_hbm.at[pl.ds(base2, PR_PW)], wt_v)
        for z in range(CR_PW // 16):
            mr16 = mr_v[pl.ds(z * 16, 16)]
            dummy = (base + z * 16 + iota) & (BATCH - 1)
            wvi_v[pl.ds(z * 16, 16)] = jnp.where(mr16 >= 0, mr16, dummy)
        for z in range(PR_PW // 16):
            wt16 = wt_v[pl.ds(z * 16, 16)]
            dummy = (base2 + z * 16 + iota) & (BATCH - 1)
            wti_v[pl.ds(z * 16, 16)] = jnp.where(wt16 >= 0, wt16, dummy)
        ha = pltpu.async_copy(mem_hbm.at[ridx_v], rows_a, sem_a)
        hb = pltpu.async_copy(wv_hbm.at[wvi_v], rows_b, sem_b)
        hc = pltpu.async_copy(
            mem_hbm.at[pl.ds(WSTART + base2, PR_PW)], rows_c, sem_a)
        hd = pltpu.async_copy(wv_hbm.at[wti_v], rows_d, sem_b)
        ha.wait()
        hb.wait()

        def cfix(i, carry):
            e = iota + i * 16
            q = lax.shift_right_logical(e, 6)
            rm = e & 63
            b = plsc.load_gather(rows_b, [q, rm])
            mr = plsc.load_gather(mr_v, [q])
            plsc.store_scatter(rows_a, [q, rm], b, mask=mr >= 0)
            return carry

        lax.fori_loop(0, CR_PW * D64 // 16, cfix, jnp.int32(0))
        pltpu.sync_copy(rows_a, c_out.at[pl.ds(base, CR_PW)])

        hc.wait()
        hd.wait()

        def pfix(i, carry):
            e = iota + i * 16
            q = lax.shift_right_logical(e, 6)
            rm = e & 63
            b = plsc.load_gather(rows_d, [q, rm])
            wt = plsc.load_gather(wt_v, [q])
            plsc.store_scatter(rows_c, [q, rm], b, mask=wt >= 0)
            return carry

        lax.fori_loop(0, PR_PW * D64 // 16, pfix, jnp.int32(0))
        pltpu.sync_copy(rows_c, pc_out.at[pl.ds(base2, PR_PW)])

    return k(mem, write_val, read_idx, m_read, w_tail)


def _sc_topk_emit(u, t, tmeta):
    mesh = plsc.VectorSubcoreMesh(core_axis_name="c", subcore_axis_name="s")

    @functools.partial(
        pl.kernel,
        out_type=jax.ShapeDtypeStruct((BATCH, K * 8), jnp.float32),
        mesh=mesh,
        compiler_params=pltpu.CompilerParams(needs_layout_passes=False),
        scratch_types=[
            pltpu.VMEM((NPAD * 8,), jnp.float32),
            pltpu.VMEM((2 * NPAD,), jnp.int32),
            pltpu.VMEM((ROWS_PER_W,), jnp.int32),
            pltpu.VMEM((LCAP,), jnp.int32),
            pltpu.VMEM((LCAP,), jnp.int32),
            pltpu.VMEM((2 * K * 8,), jnp.float32),
            pltpu.SemaphoreType.DMA,
            pltpu.SemaphoreType.DMA,
        ],
    )
    def k(u_hbm, t_hbm, tm_hbm, out_hbm,
          tm_v, u_v, t_v, lv, li, out_v, sem_in, sem_out):
        wid = lax.axis_index("s") * 2 + lax.axis_index("c")
        base = wid * ROWS_PER_W
        pltpu.sync_copy(tm_hbm, tm_v)
        pltpu.sync_copy(t_hbm.at[pl.ds(base, ROWS_PER_W)], t_v)
        pltpu.async_copy(u_hbm.at[base], u_v.at[pl.ds(0, NPAD)], sem_in)
        iota = lax.iota(jnp.int32, 16)
        lane15 = jnp.full((16,), 15, jnp.int32)

        def row_body(r, carry):
            cur = lax.rem(r, 2)
            nxt = 1 - cur

            @pl.when(r + 1 < ROWS_PER_W)
            def _():
                pltpu.async_copy(u_hbm.at[base + r + 1],
                                 u_v.at[pl.ds(nxt * NPAD, NPAD)], sem_in)

            pltpu.make_async_copy(
                u_hbm.at[base], u_v.at[pl.ds(cur * NPAD, NPAD)], sem_in).wait()
            chunk = r // 16
            lane = r - chunk * 16
            tv16 = t_v[pl.ds(chunk * 16, 16)]
            tvec = tv16[jnp.full((16,), lane, jnp.int32)]
            for z in range(8):
                lv[pl.ds(z * 16, 16)] = jnp.full((16,), -2147483648, jnp.int32)
                li[pl.ds(z * 16, 16)] = jnp.zeros((16,), jnp.int32)

            @plsc.parallel_loop(0, NVREG, 1, unroll=4,
                                carry=jnp.zeros((16,), jnp.int32))
            def _vloop(i, nv):
                x = u_v[pl.ds(cur * NPAD + i * 16, 16)]
                m = x >= tvec
                cs = plsc.cumsum(jnp.where(m, jnp.int32(1), jnp.int32(0)))
                pos = nv + cs - 1
                plsc.store_scatter(lv, [pos], x, mask=m)
                plsc.store_scatter(li, [pos], iota + i * 16, mask=m)
                return nv + cs[lane15]

            pairs = [(lv[pl.ds(z * 16, 16)], li[pl.ds(z * 16, 16)])
                     for z in range(8)]
            srt = _sort128_desc(pairs)

            @pl.when(r >= 2)
            def _():
                pltpu.make_async_copy(
                    out_v.at[pl.ds(cur * K * 8, K * 8)],
                    out_hbm.at[base], sem_out).wait()

            for kk in range(K // 16):
                mv, xv = srt[kk]
                uv = lax.bitcast_convert_type(
                    mv ^ jnp.int32(-2147483648), jnp.uint32)
                neg = ~lax.bitcast_convert_type(uv, jnp.int32)
                pos = lax.bitcast_convert_type(
                    uv & jnp.uint32(0x7FFFFFFF), jnp.int32)
                sv = jnp.where(uv >= jnp.uint32(0x80000000), pos, neg)
                val = lax.bitcast_convert_type(sv, jnp.float32)
                j8 = (iota + kk * 16) * 8 + cur * (K * 8)
                plsc.store_scatter(out_v, [j8], val)
                x8 = xv * 8
                for c in range(1, 8):
                    g = plsc.load_gather(tm_v, [x8 + c])
                    plsc.store_scatter(out_v, [j8 + c], g)
            pltpu.async_copy(out_v.at[pl.ds(cur * K * 8, K * 8)],
                             out_hbm.at[base + r], sem_out)
            return carry

        lax.fori_loop(0, ROWS_PER_W, row_body, jnp.int32(0))
        pltpu.make_async_copy(
            out_v.at[pl.ds(0, K * 8)], out_hbm.at[base], sem_out).wait()
        pltpu.make_async_copy(
            out_v.at[pl.ds(K * 8, K * 8)], out_hbm.at[base], sem_out).wait()

    return k(u, t, tmeta)


def kernel(mem, write_val, actions_table, rewards_table, write_idx, read_idx):
    start = CAP - NREC
    ids = jnp.concatenate(
        [read_idx, WSTART + jnp.arange(NPAD, dtype=jnp.int32)])
    m_all = _match(ids.reshape(NIDS, 1), write_idx.reshape(1, BATCH))
    m_all = m_all.reshape(NIDS)
    m_read, w_tail = m_all[:BATCH], m_all[BATCH:]

    mem64 = jnp.pad(mem, ((0, 0), (0, D64 - DOBS)))
    wv64 = jnp.pad(write_val, ((0, 0), (0, D64 - DOBS)))
    c, past_c_pad = _sc_build(mem64, wv64, read_idx, m_read, w_tail)
    u, t = _deltas_thresh(c, past_c_pad)
    t = t.reshape(BATCH)
    tmeta = jnp.concatenate(
        [jnp.zeros((NREC, 1), jnp.float32), actions_table[start:],
         rewards_table[start:]], axis=1)
    tmeta = jnp.concatenate(
        [jnp.zeros((PADF, 8), jnp.float32), tmeta], axis=0).reshape(-1)
    out = _sc_topk_emit(u, t, tmeta)
    return out.reshape(BATCH, K, 8)


# speedup vs baseline: 1.4839x; 1.0105x over previous
import functools

import jax
import jax.numpy as jnp
from jax import lax
from jax.experimental import pallas as pl
from jax.experimental.pallas import tpu as pltpu
from jax.experimental.pallas import tpu_sc as plsc

CAP = 100000
NREC = 5000
K = 80
DOBS = 50
BATCH = 4096

ROWS_PER_BLK = 512
NPAD = 5120
WSTART = CAP - NPAD
PADF = NPAD - NREC
D64 = 64


def _mm_thresh_body(c_ref, pc_ref, u_ref, t_ref):
    deltas = jax.lax.dot_general(
        c_ref[...], pc_ref[...], (((1,), (1,)), ((), ())),
        preferred_element_type=jnp.float32)
    s = jax.lax.bitcast_convert_type(deltas, jnp.int32)
    u = jnp.where(s < 0, ~s, s | jnp.int32(-2147483648)).astype(jnp.uint32)
    col = jax.lax.broadcasted_iota(jnp.int32, u.shape, 1)
    u = jnp.where(col >= PADF, u, jnp.uint32(0))
    u_ref[...] = jax.lax.bitcast_convert_type(u, jnp.int32) ^ jnp.int32(-2147483648)
    t = jnp.zeros((ROWS_PER_BLK, 1), jnp.uint32)
    for bit in range(31, 15, -1):
        cand = t | jnp.uint32(1 << bit)
        cnt = jnp.sum((u >= cand).astype(jnp.int32), axis=1, keepdims=True)
        t = jnp.where(cnt >= K, cand, t)
    ti = jax.lax.bitcast_convert_type(t, jnp.int32) ^ jnp.int32(-2147483648)
    t_ref[...] = ti.reshape(1, 1, ROWS_PER_BLK)


def _deltas_thresh(c, past_c_pad):
    grid = BATCH // ROWS_PER_BLK
    return pl.pallas_call(
        _mm_thresh_body,
        grid=(grid,),
        in_specs=[
            pl.BlockSpec((ROWS_PER_BLK, D64), lambda i: (i, 0)),
            pl.BlockSpec((NPAD, D64), lambda i: (0, 0)),
        ],
        out_specs=[
            pl.BlockSpec((ROWS_PER_BLK, NPAD), lambda i: (i, 0)),
            pl.BlockSpec((1, 1, ROWS_PER_BLK), lambda i: (i, 0, 0)),
        ],
        out_shape=[
            jax.ShapeDtypeStruct((BATCH, NPAD), jnp.int32),
            jax.ShapeDtypeStruct((grid, 1, ROWS_PER_BLK), jnp.int32),
        ],
    )(c, past_c_pad)


MBLK = 1024
NIDS = BATCH + NPAD


def _match_body(ids_ref, wi_ref, out_ref):
    ids = ids_ref[...]
    wi = wi_ref[...]
    m = jnp.full((MBLK, 1), -1, jnp.int32)
    cw = 1024
    for cc in range(BATCH // cw):
        eq = ids == wi[:, cc * cw:(cc + 1) * cw]
        j = jax.lax.broadcasted_iota(jnp.int32, (MBLK, cw), 1) + cc * cw
        m = jnp.maximum(m, jnp.max(jnp.where(eq, j, -1), axis=1,
                                   keepdims=True))
    out_ref[...] = m


def _match(ids_col, wi_row):
    grid = NIDS // MBLK
    return pl.pallas_call(
        _match_body,
        grid=(grid,),
        in_specs=[
            pl.BlockSpec((MBLK, 1), lambda i: (i, 0)),
            pl.BlockSpec((1, BATCH), lambda i: (0, 0)),
        ],
        out_specs=pl.BlockSpec((MBLK, 1), lambda i: (i, 0)),
        out_shape=jax.ShapeDtypeStruct((NIDS, 1), jnp.int32),
    )(ids_col, wi_row)


NW = 32
ROWS_PER_W = BATCH // NW
NVREG = NPAD // 16
LCAP = NPAD + 16


def _vsort16(p):
    v, x = p
    return plsc.sort_key_val(v, x, descending=True)


def _rev(p):
    v, x = p
    return lax.rev(v, (0,)), lax.rev(x, (0,))


def _bitonic_desc(vs):
    if len(vs) == 1:
        return [_vsort16(vs[0])]
    half = len(vs) // 2
    hi, lo = [], []
    for i in range(half):
        av, ax = vs[i]
        bv, bx = vs[i + half]
        sel = av >= bv
        hi.append((jnp.where(sel, av, bv), jnp.where(sel, ax, bx)))
        lo.append((jnp.where(sel, bv, av), jnp.where(sel, bx, ax)))
    return _bitonic_desc(hi) + _bitonic_desc(lo)


def _merge_desc(a, b):
    return _bitonic_desc(a + [_rev(p) for p in reversed(b)])


def _sort128_desc(pairs):
    runs = [[_vsort16(p)] for p in pairs]
    while len(runs) > 1:
        runs = [_merge_desc(runs[i], runs[i + 1])
                for i in range(0, len(runs), 2)]
    return runs[0]


CR_PW = BATCH // NW
PR_PW = NPAD // NW


def _sc_build(mem, write_val, read_idx, m_read, w_tail):
    mesh = plsc.VectorSubcoreMesh(core_axis_name="c", subcore_axis_name="s")

    @functools.partial(
        pl.kernel,
        out_type=[
            jax.ShapeDtypeStruct((BATCH, D64), jnp.float32),
            jax.ShapeDtypeStruct((NPAD, D64), jnp.float32),
        ],
        mesh=mesh,
        compiler_params=pltpu.CompilerParams(
            needs_layout_passes=False, use_tc_tiling_on_sc=False),
        scratch_types=[
            pltpu.VMEM((CR_PW,), jnp.int32),
            pltpu.VMEM((CR_PW,), jnp.int32),
            pltpu.VMEM((CR_PW,), jnp.int32),
            pltpu.VMEM((CR_PW, D64), jnp.float32),
            pltpu.VMEM((CR_PW, D64), jnp.float32),
            pltpu.VMEM((PR_PW,), jnp.int32),
            pltpu.VMEM((PR_PW,), jnp.int32),
            pltpu.VMEM((PR_PW, D64), jnp.float32),
            pltpu.VMEM((PR_PW, D64), jnp.float32),
            pltpu.SemaphoreType.DMA,
            pltpu.SemaphoreType.DMA,
        ],
    )
    def k(mem_hbm, wv_hbm, ridx_hbm, mr_hbm, wt_hbm, c_out, pc_out,
          ridx_v, mr_v, wvi_v, rows_a, rows_b, wt_v, wti_v, rows_c, rows_d,
          sem_a, sem_b):
        wid = lax.axis_index("s") * 2 + lax.axis_index("c")
        iota = lax.iota(jnp.int32, 16)
        base = wid * CR_PW
        base2 = wid * PR_PW
        pltpu.sync_copy(ridx_hbm.at[pl.ds(base, CR_PW)], ridx_v)
        pltpu.sync_copy(mr_hbm.at[pl.ds(base, CR_PW)], mr_v)
        pltpu.sync_copy(wt_hbm.at[pl.ds(base2, PR_PW)], wt_v)
        for z in range(CR_PW // 16):
            mr16 = mr_v[pl.ds(z * 16, 16)]
            dummy = (base + z * 16 + iota) & (BATCH - 1)
            wvi_v[pl.ds(z * 16, 16)] = jnp.where(mr16 >= 0, mr16, dummy)
        for z in range(PR_PW // 16):
            wt16 = wt_v[pl.ds(z * 16, 16)]
            dummy = (base2 + z * 16 + iota) & (BATCH - 1)
            wti_v[pl.ds(z * 16, 16)] = jnp.where(wt16 >= 0, wt16, dummy)
        ha = pltpu.async_copy(mem_hbm.at[ridx_v], rows_a, sem_a)
        hb = pltpu.async_copy(wv_hbm.at[wvi_v], rows_b, sem_b)
        hc = pltpu.async_copy(
            mem_hbm.at[pl.ds(WSTART + base2, PR_PW)], rows_c, sem_a)
        hd = pltpu.async_copy(wv_hbm.at[wti_v], rows_d, sem_b)
        ha.wait()
        hb.wait()

        def cfix(i, carry):
            e = iota + i * 16
            q = lax.shift_right_logical(e, 6)
            rm = e & 63
            b = plsc.load_gather(rows_b, [q, rm])
            mr = plsc.load_gather(mr_v, [q])
            plsc.store_scatter(rows_a, [q, rm], b, mask=mr >= 0)
            return carry

        lax.fori_loop(0, CR_PW * D64 // 16, cfix, jnp.int32(0))
        pltpu.sync_copy(rows_a, c_out.at[pl.ds(base, CR_PW)])

        hc.wait()
        hd.wait()

        def pfix(i, carry):
            e = iota + i * 16
            q = lax.shift_right_logical(e, 6)
            rm = e & 63
            b = plsc.load_gather(rows_d, [q, rm])
            wt = plsc.load_gather(wt_v, [q])
            plsc.store_scatter(rows_c, [q, rm], b, mask=wt >= 0)
            return carry

        lax.fori_loop(0, PR_PW * D64 // 16, pfix, jnp.int32(0))
        pltpu.sync_copy(rows_c, pc_out.at[pl.ds(base2, PR_PW)])

    return k(mem, write_val, read_idx, m_read, w_tail)


def _sc_topk_emit(u, t, tmeta):
    mesh = plsc.VectorSubcoreMesh(core_axis_name="c", subcore_axis_name="s")

    @functools.partial(
        pl.kernel,
        out_type=jax.ShapeDtypeStruct((BATCH, K * 8), jnp.float32),
        mesh=mesh,
        compiler_params=pltpu.CompilerParams(needs_layout_passes=False),
        scratch_types=[
            pltpu.VMEM((NPAD * 8,), jnp.float32),
            pltpu.VMEM((2 * NPAD,), jnp.int32),
            pltpu.VMEM((ROWS_PER_W,), jnp.int32),
            pltpu.VMEM((LCAP,), jnp.int32),
            pltpu.VMEM((LCAP,), jnp.int32),
            pltpu.VMEM((2 * K * 8,), jnp.float32),
            pltpu.SemaphoreType.DMA,
            pltpu.SemaphoreType.DMA,
        ],
    )
    def k(u_hbm, t_hbm, tm_hbm, out_hbm,
          tm_v, u_v, t_v, lv, li, out_v, sem_in, sem_out):
        wid = lax.axis_index("s") * 2 + lax.axis_index("c")
        base = wid * ROWS_PER_W
        pltpu.sync_copy(tm_hbm, tm_v)
        pltpu.sync_copy(t_hbm.at[pl.ds(base, ROWS_PER_W)], t_v)
        pltpu.async_copy(u_hbm.at[base], u_v.at[pl.ds(0, NPAD)], sem_in)
        iota = lax.iota(jnp.int32, 16)
        lane15 = jnp.full((16,), 15, jnp.int32)

        def row_body(r, carry):
            cur = lax.rem(r, 2)
            nxt = 1 - cur

            @pl.when(r + 1 < ROWS_PER_W)
            def _():
                pltpu.async_copy(u_hbm.at[base + r + 1],
                                 u_v.at[pl.ds(nxt * NPAD, NPAD)], sem_in)

            pltpu.make_async_copy(
                u_hbm.at[base], u_v.at[pl.ds(cur * NPAD, NPAD)], sem_in).wait()
            chunk = r // 16
            lane = r - chunk * 16
            tv16 = t_v[pl.ds(chunk * 16, 16)]
            tvec = tv16[jnp.full((16,), lane, jnp.int32)]
            for z in range(8):
                lv[pl.ds(z * 16, 16)] = jnp.full((16,), -2147483648, jnp.int32)
                li[pl.ds(z * 16, 16)] = jnp.zeros((16,), jnp.int32)

            @plsc.parallel_loop(0, NVREG, 1, unroll=4,
                                carry=jnp.zeros((16,), jnp.int32))
            def _vloop(i, nv):
                x = u_v[pl.ds(cur * NPAD + i * 16, 16)]
                m = x >= tvec
                cs = plsc.cumsum(jnp.where(m, jnp.int32(1), jnp.int32(0)))
                pos = nv + cs - 1
                plsc.store_scatter(lv, [pos], x, mask=m)
                plsc.store_scatter(li, [pos], iota + i * 16, mask=m)
                return nv + cs[lane15]

            pairs = [(lv[pl.ds(z * 16, 16)], li[pl.ds(z * 16, 16)])
                     for z in range(8)]
            srt = _sort128_desc(pairs)

            @pl.when(r >= 2)
            def _():
                pltpu.make_async_copy(
                    out_v.at[pl.ds(cur * K * 8, K * 8)],
                    out_hbm.at[base], sem_out).wait()

            for kk in range(K // 16):
                mv, xv = srt[kk]
                uv = lax.bitcast_convert_type(
                    mv ^ jnp.int32(-2147483648), jnp.uint32)
                neg = ~lax.bitcast_convert_type(uv, jnp.int32)
                pos = lax.bitcast_convert_type(
                    uv & jnp.uint32(0x7FFFFFFF), jnp.int32)
                sv = jnp.where(uv >= jnp.uint32(0x80000000), pos, neg)
                val = lax.bitcast_convert_type(sv, jnp.float32)
                j8 = (iota + kk * 16) * 8 + cur * (K * 8)
                plsc.store_scatter(out_v, [j8], val)
                x8 = xv * 8
                for c in range(1, 8):
                    g = plsc.load_gather(tm_v, [x8 + c])
                    plsc.store_scatter(out_v, [j8 + c], g)
            pltpu.async_copy(out_v.at[pl.ds(cur * K * 8, K * 8)],
                             out_hbm.at[base + r], sem_out)
            return carry

        lax.fori_loop(0, ROWS_PER_W, row_body, jnp.int32(0))
        pltpu.make_async_copy(
            out_v.at[pl.ds(0, K * 8)], out_hbm.at[base], sem_out).wait()
        pltpu.make_async_copy(
            out_v.at[pl.ds(K * 8, K * 8)], out_hbm.at[base], sem_out).wait()

    return k(u, t, tmeta)


def kernel(mem, write_val, actions_table, rewards_table, write_idx, read_idx):
    start = CAP - NREC
    ids = jnp.concatenate(
        [read_idx, WSTART + jnp.arange(NPAD, dtype=jnp.int32)])
    m_all = _match(ids.reshape(NIDS, 1), write_idx.reshape(1, BATCH))
    m_all = m_all.reshape(NIDS)
    m_read, w_tail = m_all[:BATCH], m_all[BATCH:]

    mem64 = jnp.pad(mem, ((0, 0), (0, D64 - DOBS)))
    wv64 = jnp.pad(write_val, ((0, 0), (0, D64 - DOBS)))
    c, past_c_pad = _sc_build(mem64, wv64, read_idx, m_read, w_tail)
    u, t = _deltas_thresh(c, past_c_pad)
    t = t.reshape(BATCH)
    tmeta = jnp.concatenate(
        [jnp.zeros((NREC, 1), jnp.float32), actions_table[start:],
         rewards_table[start:]], axis=1)
    tmeta = jnp.concatenate(
        [jnp.zeros((PADF, 8), jnp.float32), tmeta], axis=0).reshape(-1)
    out = _sc_topk_emit(u, t, tmeta)
    return out.reshape(BATCH, K, 8)
